# baseline (speedup 1.0000x reference)
import functools

import jax
import jax.numpy as jnp
from jax.experimental import pallas as pl
from jax.experimental.pallas import tpu as pltpu


def _round_up(x, m):
    return ((x + m - 1) // m) * m


def _proj_kernel(feat_ref, wT_ref, p_ref):
    p_ref[...] = jnp.dot(
        feat_ref[...], wT_ref[...], preferred_element_type=jnp.float32
    )


def _gather_diff_kernel(src_ref, dst_ref, p_ref, b_ref, w_ref, x_ref, *, unroll):
    te = w_ref.shape[0]
    C = w_ref.shape[1]
    bvec = b_ref[0]

    def body(i, carry):
        base = pl.multiple_of(i * unroll, unroll)
        diffs = []
        for u in range(unroll):
            e = base + u
            s = src_ref[0, 0, e]
            d = dst_ref[0, 0, e]
            diffs.append(p_ref[s] - p_ref[d] + bvec)
        cat = jnp.concatenate(diffs, axis=0)
        w_ref[pl.ds(base, unroll), :] = cat[:, :C]
        x_ref[pl.ds(base, unroll), :] = cat[:, C:]
        return carry

    jax.lax.fori_loop(0, te // unroll, body, 0)


@functools.partial(
    jax.jit, static_argnames=("node_tile", "edge_tile", "unroll"))
def _score_edges(features, src, dst, W_w, b_w, W_x, b_x, *,
                 node_tile=1024, edge_tile=2048, unroll=16):
    N, F = features.shape
    C = W_w.shape[0]
    E = src.shape[0]
    C2 = 2 * C

    W_cat_T = jnp.concatenate([W_w, W_x], axis=0).T.astype(jnp.float32)
    b_cat = jnp.concatenate([b_w, b_x]).reshape(1, 1, C2).astype(jnp.float32)

    tn = min(node_tile, _round_up(N, 8))
    N_pad = _round_up(N, tn)
    feats = features.astype(jnp.float32)
    if N_pad != N:
        feats = jnp.pad(feats, ((0, N_pad - N), (0, 0)))

    P = pl.pallas_call(
        _proj_kernel,
        out_shape=jax.ShapeDtypeStruct((N_pad, C2), jnp.float32),
        grid_spec=pltpu.PrefetchScalarGridSpec(
            num_scalar_prefetch=0,
            grid=(N_pad // tn,),
            in_specs=[pl.BlockSpec((tn, F), lambda i: (i, 0)),
                      pl.BlockSpec((F, C2), lambda i: (0, 0))],
            out_specs=pl.BlockSpec((tn, C2), lambda i: (i, 0)),
        ),
        compiler_params=pltpu.CompilerParams(
            dimension_semantics=("parallel",)),
    )(feats, W_cat_T)

    te = min(edge_tile, _round_up(E, unroll))
    E_pad = _round_up(E, te)
    if E_pad != E:
        pad = (0, E_pad - E)
        src = jnp.pad(src, pad)
        dst = jnp.pad(dst, pad)
    G = E_pad // te

    P3 = P.reshape(N_pad, 1, C2)
    src3 = src.reshape(G, 1, te)
    dst3 = dst.reshape(G, 1, te)

    idx_spec = pl.BlockSpec((1, 1, te), lambda i: (i, 0, 0),
                            memory_space=pltpu.SMEM)
    out_spec = pl.BlockSpec((te, C), lambda i: (i, 0))
    w2, x2 = pl.pallas_call(
        functools.partial(_gather_diff_kernel, unroll=unroll),
        out_shape=(
            jax.ShapeDtypeStruct((E_pad, C), jnp.float32),
            jax.ShapeDtypeStruct((E_pad, C), jnp.float32),
        ),
        grid_spec=pltpu.PrefetchScalarGridSpec(
            num_scalar_prefetch=0,
            grid=(G,),
            in_specs=[
                idx_spec,
                idx_spec,
                pl.BlockSpec((N_pad, 1, C2), lambda i: (0, 0, 0)),
                pl.BlockSpec((1, 1, C2), lambda i: (0, 0, 0)),
            ],
            out_specs=(out_spec, out_spec),
        ),
        compiler_params=pltpu.CompilerParams(
            dimension_semantics=("parallel",)),
    )(src3, dst3, P3, b_cat)

    return w2[:E], x2[:E]


def kernel(features, src, dst, W_w, b_w, W_x, b_x):
    return _score_edges(features, src, dst, W_w, b_w, W_x, b_x)


# speedup vs baseline: 1.0670x; 1.0670x over previous
import functools

import jax
import jax.numpy as jnp
from jax.experimental import pallas as pl
from jax.experimental.pallas import tpu as pltpu


def _round_up(x, m):
    return ((x + m - 1) // m) * m


def _proj_kernel(feat_ref, wT_ref, p_ref):
    p_ref[...] = jnp.dot(
        feat_ref[...], wT_ref[...], preferred_element_type=jnp.float32
    )


def _gather_diff_kernel(src_ref, dst_ref, p_ref, b_ref, w_ref, x_ref, *, unroll):
    te = w_ref.shape[0]
    C = w_ref.shape[1]
    bvec = b_ref[0]

    def body(i, carry):
        base = pl.multiple_of(i * unroll, unroll)
        diffs = []
        for u in range(unroll):
            e = base + u
            s = src_ref[0, 0, e]
            d = dst_ref[0, 0, e]
            diffs.append(p_ref[s] - p_ref[d] + bvec)
        cat = jnp.concatenate(diffs, axis=0)
        w_ref[pl.ds(base, unroll), :] = cat[:, :C]
        x_ref[pl.ds(base, unroll), :] = cat[:, C:]
        return carry

    jax.lax.fori_loop(0, te // unroll, body, 0)


@functools.partial(
    jax.jit, static_argnames=("node_tile", "edge_tile", "unroll"))
def _score_edges(features, src, dst, W_w, b_w, W_x, b_x, *,
                 node_tile=1024, edge_tile=4096, unroll=32):
    N, F = features.shape
    C = W_w.shape[0]
    E = src.shape[0]
    C2 = 2 * C

    W_cat_T = jnp.concatenate([W_w, W_x], axis=0).T.astype(jnp.float32)
    b_cat = jnp.concatenate([b_w, b_x]).reshape(1, 1, C2).astype(jnp.float32)

    tn = min(node_tile, _round_up(N, 8))
    N_pad = _round_up(N, tn)
    feats = features.astype(jnp.float32)
    if N_pad != N:
        feats = jnp.pad(feats, ((0, N_pad - N), (0, 0)))

    P = pl.pallas_call(
        _proj_kernel,
        out_shape=jax.ShapeDtypeStruct((N_pad, C2), jnp.float32),
        grid_spec=pltpu.PrefetchScalarGridSpec(
            num_scalar_prefetch=0,
            grid=(N_pad // tn,),
            in_specs=[pl.BlockSpec((tn, F), lambda i: (i, 0)),
                      pl.BlockSpec((F, C2), lambda i: (0, 0))],
            out_specs=pl.BlockSpec((tn, C2), lambda i: (i, 0)),
        ),
        compiler_params=pltpu.CompilerParams(
            dimension_semantics=("parallel",)),
    )(feats, W_cat_T)

    te = min(edge_tile, _round_up(E, unroll))
    E_pad = _round_up(E, te)
    if E_pad != E:
        pad = (0, E_pad - E)
        src = jnp.pad(src, pad)
        dst = jnp.pad(dst, pad)
    G = E_pad // te

    P3 = P.reshape(N_pad, 1, C2)
    src3 = src.reshape(G, 1, te)
    dst3 = dst.reshape(G, 1, te)

    idx_spec = pl.BlockSpec((1, 1, te), lambda i: (i, 0, 0),
                            memory_space=pltpu.SMEM)
    out_spec = pl.BlockSpec((te, C), lambda i: (i, 0))
    w2, x2 = pl.pallas_call(
        functools.partial(_gather_diff_kernel, unroll=unroll),
        out_shape=(
            jax.ShapeDtypeStruct((E_pad, C), jnp.float32),
            jax.ShapeDtypeStruct((E_pad, C), jnp.float32),
        ),
        grid_spec=pltpu.PrefetchScalarGridSpec(
            num_scalar_prefetch=0,
            grid=(G,),
            in_specs=[
                idx_spec,
                idx_spec,
                pl.BlockSpec((N_pad, 1, C2), lambda i: (0, 0, 0)),
                pl.BlockSpec((1, 1, C2), lambda i: (0, 0, 0)),
            ],
            out_specs=(out_spec, out_spec),
        ),
        compiler_params=pltpu.CompilerParams(
            dimension_semantics=("parallel",)),
    )(src3, dst3, P3, b_cat)

    return w2[:E], x2[:E]


def kernel(features, src, dst, W_w, b_w, W_x, b_x):
    return _score_edges(features, src, dst, W_w, b_w, W_x, b_x)
